# baseline (speedup 1.0000x reference)
import functools

import jax
import jax.numpy as jnp
from jax import lax
from jax.experimental import pallas as pl
from jax.experimental.pallas import tpu as pltpu
from jax.experimental.pallas import tpu_sc as plsc

D = 768
FF = 1536
E = 8
K = 2
T = 2048
C = 640
TRASH = E * C
BUF_ROWS = E * C + 8
FB = 512
NJ = FF // FB
TB = 256
NB = T // TB


def _router_body(x_ref, wg_ref, sd_ref, sc_ref, w0x_ref, w1x_ref,
                 oh1_ref, oh2_ref, c1_ref, c2_ref):
    logitsT = lax.dot_general(wg_ref[...], x_ref[...],
                              (((0,), (1,)), ((), ())),
                              preferred_element_type=jnp.float32)
    mx = jnp.max(logitsT, axis=0, keepdims=True)
    ex = jnp.exp(logitsT - mx)
    probs = ex / jnp.sum(ex, axis=0, keepdims=True)
    iota = lax.broadcasted_iota(jnp.int32, (E, T), 0)
    m1 = jnp.max(probs, axis=0, keepdims=True)
    i1 = jnp.min(jnp.where(probs >= m1, iota, E), axis=0, keepdims=True)
    oh1 = iota == i1
    probsm = jnp.where(oh1, -jnp.inf, probs)
    m2 = jnp.max(probsm, axis=0, keepdims=True)
    i2 = jnp.min(jnp.where(probsm >= m2, iota, E), axis=0, keepdims=True)
    oh2 = iota == i2
    denom = m1 + m2 + 1e-9
    g1 = m1 / denom
    g2 = m2 / denom

    oh1_ref[...] = oh1.astype(jnp.float32)
    oh2_ref[...] = oh2.astype(jnp.float32)
    rows = lax.broadcasted_iota(jnp.int32, (TB, TB), 0)
    cols = lax.broadcasted_iota(jnp.int32, (TB, TB), 1)
    triu = (rows <= cols).astype(jnp.float32)

    def body(b, carries):
        car1, car2 = carries
        blk1 = oh1_ref[:, pl.ds(b * TB, TB)]
        blk2 = oh2_ref[:, pl.ds(b * TB, TB)]
        cs1 = jnp.dot(blk1, triu, preferred_element_type=jnp.float32) + car1
        cs2 = jnp.dot(blk2, triu, preferred_element_type=jnp.float32) + car2
        c1_ref[:, pl.ds(b * TB, TB)] = jnp.sum(cs1 * blk1, axis=0, keepdims=True)
        c2_ref[:, pl.ds(b * TB, TB)] = jnp.sum(cs2 * blk2, axis=0, keepdims=True)
        return (car1 + jnp.sum(blk1, axis=1, keepdims=True),
                car2 + jnp.sum(blk2, axis=1, keepdims=True))

    zero8 = jnp.zeros((E, 1), jnp.float32)
    tot1, _ = lax.fori_loop(0, NB, body, (zero8, zero8))

    pos0 = c1_ref[...].astype(jnp.int32) - 1
    tot1_own = jnp.sum(tot1 * oh2_ref[...], axis=0, keepdims=True)
    pos1 = (c2_ref[...] + tot1_own).astype(jnp.int32) - 1
    keep0 = pos0 < C
    keep1 = pos1 < C
    slotc0 = i1 * C + jnp.where(keep0, pos0, 0)
    slotc1 = i2 * C + jnp.where(keep1, pos1, 0)
    sd_ref[0:1, :] = jnp.where(keep0, slotc0, TRASH)
    sd_ref[1:2, :] = jnp.where(keep1, slotc1, TRASH)
    sc_ref[0:1, :] = slotc0
    sc_ref[1:2, :] = slotc1
    w0 = jnp.where(keep0, g1, 0.0)
    w1 = jnp.where(keep1, g2, 0.0)
    ones16 = jnp.ones((1, 16), jnp.float32)
    w0x_ref[...] = jnp.transpose(w0) * ones16
    w1x_ref[...] = jnp.transpose(w1) * ones16


def _router(xf, Wg):
    return pl.pallas_call(
        _router_body,
        out_shape=[jax.ShapeDtypeStruct((K, T), jnp.int32),
                   jax.ShapeDtypeStruct((K, T), jnp.int32),
                   jax.ShapeDtypeStruct((T, 16), jnp.float32),
                   jax.ShapeDtypeStruct((T, 16), jnp.float32)],
        scratch_shapes=[pltpu.VMEM((E, T), jnp.float32),
                        pltpu.VMEM((E, T), jnp.float32),
                        pltpu.VMEM((1, T), jnp.float32),
                        pltpu.VMEM((1, T), jnp.float32)],
    )(xf, Wg)


def _dispatch(xf, slots):
    info = plsc.get_sparse_core_info()
    nc, ns = info.num_cores, info.num_subcores
    nw = nc * ns
    ch = (K * T) // nw
    mesh = plsc.VectorSubcoreMesh(core_axis_name="c", subcore_axis_name="s")

    nch = 4
    cs = ch // nch

    @functools.partial(
        pl.kernel, mesh=mesh,
        out_type=jax.ShapeDtypeStruct((BUF_ROWS, D), jnp.float32),
        scratch_types=[pltpu.VMEM((cs,), jnp.int32),
                       pltpu.VMEM((cs,), jnp.int32),
                       pltpu.VMEM((cs,), jnp.int32),
                       pltpu.VMEM((cs,), jnp.int32),
                       pltpu.VMEM((ch, D), jnp.float32),
                       pltpu.SemaphoreType.DMA((4,)),
                       pltpu.SemaphoreType.DMA((4,)),
                       pltpu.SemaphoreType.DMA((4,))],
    )
    def k(x_hbm, sl_hbm, buf_hbm, i0, i1, i2, i3, rows_v, semi, semx, sems):
        wid = lax.axis_index("s") * nc + lax.axis_index("c")
        base = wid * ch
        kk = base // T
        off = base - kk * T
        idx_refs = [i0, i1, i2, i3]
        icps, xcps = [], []
        for q in range(nch):
            icps.append(pltpu.async_copy(
                sl_hbm.at[kk, pl.ds(off + q * cs, cs)], idx_refs[q],
                semi.at[q]))
            xcps.append(pltpu.async_copy(
                x_hbm.at[pl.ds(off + q * cs, cs)],
                rows_v.at[pl.ds(q * cs, cs)], semx.at[q]))
        scps = []
        for q in range(nch):
            icps[q].wait()
            xcps[q].wait()
            scps.append(pltpu.async_copy(
                rows_v.at[pl.ds(q * cs, cs)], buf_hbm.at[idx_refs[q]],
                sems.at[q]))
        for cp in scps:
            cp.wait()

    return k(xf, slots)


def _ffn_body(x_ref, wi_ref, wo_ref, o_ref):
    xb = x_ref[...].astype(jnp.bfloat16)
    wi = wi_ref[0].astype(jnp.bfloat16)
    a = jnp.dot(xb, wi[:, :FF], preferred_element_type=jnp.float32)
    g = jnp.dot(xb, wi[:, FF:], preferred_element_type=jnp.float32)
    act = jax.nn.gelu(a) * g
    o_ref[...] = jnp.dot(act.astype(jnp.bfloat16),
                         wo_ref[0].astype(jnp.bfloat16),
                         preferred_element_type=jnp.float32)


def _ffn(buf, W_in, W_out):
    return pl.pallas_call(
        _ffn_body,
        grid=(E,),
        in_specs=[
            pl.BlockSpec((C, D), lambda e: (e, 0)),
            pl.BlockSpec((1, D, 2 * FF), lambda e: (e, 0, 0)),
            pl.BlockSpec((1, FF, D), lambda e: (e, 0, 0)),
        ],
        out_specs=pl.BlockSpec((C, D), lambda e: (e, 0)),
        out_shape=jax.ShapeDtypeStruct((E * C, D), jnp.float32),
        compiler_params=pltpu.CompilerParams(
            dimension_semantics=("arbitrary",)),
    )(buf, W_in, W_out)


def _combine(out_rows, slots, w0x, w1x):
    info = plsc.get_sparse_core_info()
    nc, ns = info.num_cores, info.num_subcores
    nw = nc * ns
    ct = T // nw
    nch = 4
    cs = ct // nch
    mesh = plsc.VectorSubcoreMesh(core_axis_name="c", subcore_axis_name="s")

    @functools.partial(
        pl.kernel, mesh=mesh,
        out_type=jax.ShapeDtypeStruct((T, D), jnp.float32),
        scratch_types=[pltpu.VMEM((ct,), jnp.int32),
                       pltpu.VMEM((ct,), jnp.int32),
                       pltpu.VMEM((ct, 16), jnp.float32),
                       pltpu.VMEM((ct, 16), jnp.float32),
                       pltpu.VMEM((ct, D), jnp.float32),
                       pltpu.VMEM((ct, D), jnp.float32),
                       pltpu.SemaphoreType.DMA((4,)),
                       pltpu.SemaphoreType.DMA((4,)),
                       pltpu.SemaphoreType.DMA((4,))],
    )
    def k(o_hbm, sl_hbm, w0_hbm, w1_hbm, y_hbm,
          idx0_v, idx1_v, w0_v, w1_v, r0_v, r1_v, sem0, sem1, semy):
        wid = lax.axis_index("s") * nc + lax.axis_index("c")
        tb = wid * ct
        pltpu.sync_copy(sl_hbm.at[0, pl.ds(tb, ct)], idx0_v)
        pltpu.sync_copy(sl_hbm.at[1, pl.ds(tb, ct)], idx1_v)
        pltpu.sync_copy(w0_hbm.at[pl.ds(tb, ct)], w0_v)
        pltpu.sync_copy(w1_hbm.at[pl.ds(tb, ct)], w1_v)
        cps = []
        for q in range(nch):
            qs = pl.ds(q * cs, cs)
            cps.append(pltpu.async_copy(o_hbm.at[idx0_v.at[qs]],
                                        r0_v.at[qs], sem0.at[q]))
            cps.append(pltpu.async_copy(o_hbm.at[idx1_v.at[qs]],
                                        r1_v.at[qs], sem1.at[q]))

        def row(r, carry):
            w0 = w0_v[r, :]
            w1 = w1_v[r, :]
            for c in range(D // 16):
                sl = pl.ds(c * 16, 16)
                r0_v[r, sl] = w0 * r0_v[r, sl] + w1 * r1_v[r, sl]
            return carry

        ycps = []
        for q in range(nch):
            cps[2 * q].wait()
            cps[2 * q + 1].wait()
            lax.fori_loop(q * cs, (q + 1) * cs, row, 0)
            qs = pl.ds(q * cs, cs)
            ycps.append(pltpu.async_copy(
                r0_v.at[qs], y_hbm.at[pl.ds(tb + q * cs, cs)], semy.at[q]))
        for cp in ycps:
            cp.wait()

    return k(out_rows, slots, w0x, w1x)


def kernel(hidden_states, Wg, W_in, W_out):
    B, S, Dm = hidden_states.shape
    xf = hidden_states.reshape(T, D)
    sd, sc_, w0x, w1x = _router(xf, Wg)
    buf = _dispatch(xf, sd)
    out = _ffn(buf, W_in, W_out)
    y = _combine(out, sc_, w0x, w1x)
    return y.reshape(B, S, Dm)


# speedup vs baseline: 1.0393x; 1.0393x over previous
import functools

import jax
import jax.numpy as jnp
from jax import lax
from jax.experimental import pallas as pl
from jax.experimental.pallas import tpu as pltpu
from jax.experimental.pallas import tpu_sc as plsc

D = 768
FF = 1536
E = 8
K = 2
T = 2048
C = 640
TRASH = E * C
BUF_ROWS = E * C + 8
FB = 512
NJ = FF // FB
TB = 256
NB = T // TB


def _router_body(x_ref, wg_ref, sd_ref, sc_ref, w0x_ref, w1x_ref,
                 oh1_ref, oh2_ref, c1_ref, c2_ref):
    logitsT = lax.dot_general(wg_ref[...], x_ref[...],
                              (((0,), (1,)), ((), ())),
                              preferred_element_type=jnp.float32)
    mx = jnp.max(logitsT, axis=0, keepdims=True)
    ex = jnp.exp(logitsT - mx)
    probs = ex / jnp.sum(ex, axis=0, keepdims=True)
    iota = lax.broadcasted_iota(jnp.int32, (E, T), 0)
    m1 = jnp.max(probs, axis=0, keepdims=True)
    i1 = jnp.min(jnp.where(probs >= m1, iota, E), axis=0, keepdims=True)
    oh1 = iota == i1
    probsm = jnp.where(oh1, -jnp.inf, probs)
    m2 = jnp.max(probsm, axis=0, keepdims=True)
    i2 = jnp.min(jnp.where(probsm >= m2, iota, E), axis=0, keepdims=True)
    oh2 = iota == i2
    denom = m1 + m2 + 1e-9
    g1 = m1 / denom
    g2 = m2 / denom

    oh1_ref[...] = oh1.astype(jnp.float32)
    oh2_ref[...] = oh2.astype(jnp.float32)
    rows = lax.broadcasted_iota(jnp.int32, (TB, TB), 0)
    cols = lax.broadcasted_iota(jnp.int32, (TB, TB), 1)
    triu = (rows <= cols).astype(jnp.float32)

    def body(b, carries):
        car1, car2 = carries
        blk1 = oh1_ref[:, pl.ds(b * TB, TB)]
        blk2 = oh2_ref[:, pl.ds(b * TB, TB)]
        cs1 = jnp.dot(blk1, triu, preferred_element_type=jnp.float32) + car1
        cs2 = jnp.dot(blk2, triu, preferred_element_type=jnp.float32) + car2
        c1_ref[:, pl.ds(b * TB, TB)] = jnp.sum(cs1 * blk1, axis=0, keepdims=True)
        c2_ref[:, pl.ds(b * TB, TB)] = jnp.sum(cs2 * blk2, axis=0, keepdims=True)
        return (car1 + jnp.sum(blk1, axis=1, keepdims=True),
                car2 + jnp.sum(blk2, axis=1, keepdims=True))

    zero8 = jnp.zeros((E, 1), jnp.float32)
    tot1, _ = lax.fori_loop(0, NB, body, (zero8, zero8))

    pos0 = c1_ref[...].astype(jnp.int32) - 1
    tot1_own = jnp.sum(tot1 * oh2_ref[...], axis=0, keepdims=True)
    pos1 = (c2_ref[...] + tot1_own).astype(jnp.int32) - 1
    keep0 = pos0 < C
    keep1 = pos1 < C
    slotc0 = i1 * C + jnp.where(keep0, pos0, 0)
    slotc1 = i2 * C + jnp.where(keep1, pos1, 0)
    sd_ref[0:1, :] = jnp.where(keep0, slotc0, TRASH)
    sd_ref[1:2, :] = jnp.where(keep1, slotc1, TRASH)
    sc_ref[0:1, :] = slotc0
    sc_ref[1:2, :] = slotc1
    w0 = jnp.where(keep0, g1, 0.0)
    w1 = jnp.where(keep1, g2, 0.0)
    ones16 = jnp.ones((1, 16), jnp.float32)
    w0x_ref[...] = jnp.transpose(w0) * ones16
    w1x_ref[...] = jnp.transpose(w1) * ones16


def _router(xf, Wg):
    return pl.pallas_call(
        _router_body,
        out_shape=[jax.ShapeDtypeStruct((K, T), jnp.int32),
                   jax.ShapeDtypeStruct((K, T), jnp.int32),
                   jax.ShapeDtypeStruct((T, 16), jnp.float32),
                   jax.ShapeDtypeStruct((T, 16), jnp.float32)],
        scratch_shapes=[pltpu.VMEM((E, T), jnp.float32),
                        pltpu.VMEM((E, T), jnp.float32),
                        pltpu.VMEM((1, T), jnp.float32),
                        pltpu.VMEM((1, T), jnp.float32)],
    )(xf, Wg)


def _dispatch(xf, slots):
    info = plsc.get_sparse_core_info()
    nc, ns = info.num_cores, info.num_subcores
    nw = nc * ns
    ch = (K * T) // nw
    mesh = plsc.VectorSubcoreMesh(core_axis_name="c", subcore_axis_name="s")

    @functools.partial(
        pl.kernel, mesh=mesh,
        out_type=jax.ShapeDtypeStruct((BUF_ROWS, D), jnp.float32),
        scratch_types=[pltpu.VMEM((ch,), jnp.int32),
                       pltpu.VMEM((ch, D), jnp.float32),
                       pltpu.SemaphoreType.DMA,
                       pltpu.SemaphoreType.DMA],
    )
    def k(x_hbm, sl_hbm, buf_hbm, idx_v, rows_v, sem, sem2):
        wid = lax.axis_index("s") * nc + lax.axis_index("c")
        base = wid * ch
        kk = base // T
        off = base - kk * T
        cp0 = pltpu.async_copy(sl_hbm.at[kk, pl.ds(off, ch)], idx_v, sem)
        cp1 = pltpu.async_copy(x_hbm.at[pl.ds(off, ch)], rows_v, sem2)
        cp0.wait()
        cp1.wait()
        pltpu.async_copy(rows_v, buf_hbm.at[idx_v], sem).wait()

    return k(xf, slots)


def _ffn_body(x_ref, wi_ref, wo_ref, o_ref):
    xb = x_ref[...].astype(jnp.bfloat16)
    wi = wi_ref[0].astype(jnp.bfloat16)
    a = jnp.dot(xb, wi[:, :FF], preferred_element_type=jnp.float32)
    g = jnp.dot(xb, wi[:, FF:], preferred_element_type=jnp.float32)
    act = jax.nn.gelu(a) * g
    o_ref[...] = jnp.dot(act.astype(jnp.bfloat16),
                         wo_ref[0].astype(jnp.bfloat16),
                         preferred_element_type=jnp.float32)


def _ffn(buf, W_in, W_out):
    return pl.pallas_call(
        _ffn_body,
        grid=(E,),
        in_specs=[
            pl.BlockSpec((C, D), lambda e: (e, 0)),
            pl.BlockSpec((1, D, 2 * FF), lambda e: (e, 0, 0)),
            pl.BlockSpec((1, FF, D), lambda e: (e, 0, 0)),
        ],
        out_specs=pl.BlockSpec((C, D), lambda e: (e, 0)),
        out_shape=jax.ShapeDtypeStruct((E * C, D), jnp.float32),
        compiler_params=pltpu.CompilerParams(
            dimension_semantics=("arbitrary",)),
    )(buf, W_in, W_out)


def _combine(out_rows, slots, w0x, w1x):
    info = plsc.get_sparse_core_info()
    nc, ns = info.num_cores, info.num_subcores
    nw = nc * ns
    ct = T // nw
    nch = 4
    cs = ct // nch
    mesh = plsc.VectorSubcoreMesh(core_axis_name="c", subcore_axis_name="s")

    @functools.partial(
        pl.kernel, mesh=mesh,
        out_type=jax.ShapeDtypeStruct((T, D), jnp.float32),
        scratch_types=[pltpu.VMEM((ct,), jnp.int32),
                       pltpu.VMEM((ct,), jnp.int32),
                       pltpu.VMEM((ct, 16), jnp.float32),
                       pltpu.VMEM((ct, 16), jnp.float32),
                       pltpu.VMEM((ct, D), jnp.float32),
                       pltpu.VMEM((ct, D), jnp.float32),
                       pltpu.SemaphoreType.DMA((4,)),
                       pltpu.SemaphoreType.DMA((4,)),
                       pltpu.SemaphoreType.DMA((4,)),
                       pltpu.SemaphoreType.DMA((4,))],
    )
    def k(o_hbm, sl_hbm, w0_hbm, w1_hbm, y_hbm,
          idx0_v, idx1_v, w0_v, w1_v, r0_v, r1_v, sem0, sem1, semy, sems):
        wid = lax.axis_index("s") * nc + lax.axis_index("c")
        tb = wid * ct
        ci0 = pltpu.async_copy(sl_hbm.at[0, pl.ds(tb, ct)], idx0_v, sems.at[0])
        ci1 = pltpu.async_copy(sl_hbm.at[1, pl.ds(tb, ct)], idx1_v, sems.at[1])
        cw0 = pltpu.async_copy(w0_hbm.at[pl.ds(tb, ct)], w0_v, sems.at[2])
        cw1 = pltpu.async_copy(w1_hbm.at[pl.ds(tb, ct)], w1_v, sems.at[3])
        ci0.wait()
        ci1.wait()
        cps = []
        for q in range(nch):
            qs = pl.ds(q * cs, cs)
            cps.append(pltpu.async_copy(o_hbm.at[idx0_v.at[qs]],
                                        r0_v.at[qs], sem0.at[q]))
            cps.append(pltpu.async_copy(o_hbm.at[idx1_v.at[qs]],
                                        r1_v.at[qs], sem1.at[q]))
        cw0.wait()
        cw1.wait()

        def row(r, carry):
            w0 = w0_v[r, :]
            w1 = w1_v[r, :]
            for c in range(D // 16):
                sl = pl.ds(c * 16, 16)
                r0_v[r, sl] = w0 * r0_v[r, sl] + w1 * r1_v[r, sl]
            return carry

        ycps = []
        for q in range(nch):
            cps[2 * q].wait()
            cps[2 * q + 1].wait()
            lax.fori_loop(q * cs, (q + 1) * cs, row, 0)
            qs = pl.ds(q * cs, cs)
            ycps.append(pltpu.async_copy(
                r0_v.at[qs], y_hbm.at[pl.ds(tb + q * cs, cs)], semy.at[q]))
        for cp in ycps:
            cp.wait()

    return k(out_rows, slots, w0x, w1x)


def kernel(hidden_states, Wg, W_in, W_out):
    B, S, Dm = hidden_states.shape
    xf = hidden_states.reshape(T, D)
    sd, sc_, w0x, w1x = _router(xf, Wg)
    buf = _dispatch(xf, sd)
    out = _ffn(buf, W_in, W_out)
    y = _combine(out, sc_, w0x, w1x)
    return y.reshape(B, S, Dm)


# speedup vs baseline: 1.0415x; 1.0021x over previous
import functools

import jax
import jax.numpy as jnp
from jax import lax
from jax.experimental import pallas as pl
from jax.experimental.pallas import tpu as pltpu
from jax.experimental.pallas import tpu_sc as plsc

D = 768
FF = 1536
E = 8
K = 2
T = 2048
C = 640
TRASH = E * C
BUF_ROWS = E * C + 8
FB = 512
NJ = FF // FB
TB = 256
NB = T // TB


def _router_body(x_ref, wg_ref, sd_ref, sc_ref, w0x_ref, w1x_ref,
                 oh1_ref, oh2_ref, c1_ref, c2_ref):
    logitsT = lax.dot_general(wg_ref[...], x_ref[...],
                              (((0,), (1,)), ((), ())),
                              preferred_element_type=jnp.float32)
    mx = jnp.max(logitsT, axis=0, keepdims=True)
    ex = jnp.exp(logitsT - mx)
    probs = ex / jnp.sum(ex, axis=0, keepdims=True)
    iota = lax.broadcasted_iota(jnp.int32, (E, T), 0)
    m1 = jnp.max(probs, axis=0, keepdims=True)
    i1 = jnp.min(jnp.where(probs >= m1, iota, E), axis=0, keepdims=True)
    oh1 = iota == i1
    probsm = jnp.where(oh1, -jnp.inf, probs)
    m2 = jnp.max(probsm, axis=0, keepdims=True)
    i2 = jnp.min(jnp.where(probsm >= m2, iota, E), axis=0, keepdims=True)
    oh2 = iota == i2
    denom = m1 + m2 + 1e-9
    g1 = m1 / denom
    g2 = m2 / denom

    oh1_ref[...] = oh1.astype(jnp.float32)
    oh2_ref[...] = oh2.astype(jnp.float32)
    rows = lax.broadcasted_iota(jnp.int32, (TB, TB), 0)
    cols = lax.broadcasted_iota(jnp.int32, (TB, TB), 1)
    triu = (rows <= cols).astype(jnp.float32)

    def body(b, carries):
        car1, car2 = carries
        blk1 = oh1_ref[:, pl.ds(b * TB, TB)]
        blk2 = oh2_ref[:, pl.ds(b * TB, TB)]
        cs1 = jnp.dot(blk1, triu, preferred_element_type=jnp.float32) + car1
        cs2 = jnp.dot(blk2, triu, preferred_element_type=jnp.float32) + car2
        c1_ref[:, pl.ds(b * TB, TB)] = jnp.sum(cs1 * blk1, axis=0, keepdims=True)
        c2_ref[:, pl.ds(b * TB, TB)] = jnp.sum(cs2 * blk2, axis=0, keepdims=True)
        return (car1 + jnp.sum(blk1, axis=1, keepdims=True),
                car2 + jnp.sum(blk2, axis=1, keepdims=True))

    zero8 = jnp.zeros((E, 1), jnp.float32)
    tot1, _ = lax.fori_loop(0, NB, body, (zero8, zero8))

    pos0 = c1_ref[...].astype(jnp.int32) - 1
    tot1_own = jnp.sum(tot1 * oh2_ref[...], axis=0, keepdims=True)
    pos1 = (c2_ref[...] + tot1_own).astype(jnp.int32) - 1
    keep0 = pos0 < C
    keep1 = pos1 < C
    slotc0 = i1 * C + jnp.where(keep0, pos0, 0)
    slotc1 = i2 * C + jnp.where(keep1, pos1, 0)
    sd_ref[0:1, :] = jnp.where(keep0, slotc0, TRASH)
    sd_ref[1:2, :] = jnp.where(keep1, slotc1, TRASH)
    sc_ref[0:1, :] = slotc0
    sc_ref[1:2, :] = slotc1
    w0 = jnp.where(keep0, g1, 0.0)
    w1 = jnp.where(keep1, g2, 0.0)
    ones16 = jnp.ones((1, 16), jnp.float32)
    w0x_ref[...] = jnp.transpose(w0) * ones16
    w1x_ref[...] = jnp.transpose(w1) * ones16


def _router(xf, Wg):
    return pl.pallas_call(
        _router_body,
        out_shape=[jax.ShapeDtypeStruct((K, T), jnp.int32),
                   jax.ShapeDtypeStruct((K, T), jnp.int32),
                   jax.ShapeDtypeStruct((T, 16), jnp.float32),
                   jax.ShapeDtypeStruct((T, 16), jnp.float32)],
        scratch_shapes=[pltpu.VMEM((E, T), jnp.float32),
                        pltpu.VMEM((E, T), jnp.float32),
                        pltpu.VMEM((1, T), jnp.float32),
                        pltpu.VMEM((1, T), jnp.float32)],
    )(xf, Wg)


def _dispatch(xf, slots):
    info = plsc.get_sparse_core_info()
    nc, ns = info.num_cores, info.num_subcores
    nw = nc * ns
    ch = (K * T) // nw
    mesh = plsc.VectorSubcoreMesh(core_axis_name="c", subcore_axis_name="s")

    @functools.partial(
        pl.kernel, mesh=mesh,
        out_type=jax.ShapeDtypeStruct((BUF_ROWS, D), jnp.float32),
        scratch_types=[pltpu.VMEM((ch,), jnp.int32),
                       pltpu.VMEM((ch, D), jnp.float32),
                       pltpu.SemaphoreType.DMA,
                       pltpu.SemaphoreType.DMA],
    )
    def k(x_hbm, sl_hbm, buf_hbm, idx_v, rows_v, sem, sem2):
        wid = lax.axis_index("s") * nc + lax.axis_index("c")
        base = wid * ch
        kk = base // T
        off = base - kk * T
        cp0 = pltpu.async_copy(sl_hbm.at[kk, pl.ds(off, ch)], idx_v, sem)
        cp1 = pltpu.async_copy(x_hbm.at[pl.ds(off, ch)], rows_v, sem2)
        cp0.wait()
        cp1.wait()
        pltpu.async_copy(rows_v, buf_hbm.at[idx_v], sem).wait()

    return k(xf, slots)


def _ffn_body(x_ref, wi_ref, wo_ref, o_ref):
    xb = x_ref[...].astype(jnp.bfloat16)
    wi = wi_ref[0].astype(jnp.bfloat16)
    a = jnp.dot(xb, wi[:, :FF], preferred_element_type=jnp.float32)
    g = jnp.dot(xb, wi[:, FF:], preferred_element_type=jnp.float32)
    act = jax.nn.gelu(a.astype(jnp.bfloat16)) * g.astype(jnp.bfloat16)
    o_ref[...] = jnp.dot(act, wo_ref[0].astype(jnp.bfloat16),
                         preferred_element_type=jnp.float32)


def _ffn(buf, W_in, W_out):
    return pl.pallas_call(
        _ffn_body,
        grid=(E,),
        in_specs=[
            pl.BlockSpec((C, D), lambda e: (e, 0)),
            pl.BlockSpec((1, D, 2 * FF), lambda e: (e, 0, 0)),
            pl.BlockSpec((1, FF, D), lambda e: (e, 0, 0)),
        ],
        out_specs=pl.BlockSpec((C, D), lambda e: (e, 0)),
        out_shape=jax.ShapeDtypeStruct((E * C, D), jnp.float32),
        compiler_params=pltpu.CompilerParams(
            dimension_semantics=("arbitrary",)),
    )(buf, W_in, W_out)


def _combine(out_rows, slots, w0x, w1x):
    info = plsc.get_sparse_core_info()
    nc, ns = info.num_cores, info.num_subcores
    nw = nc * ns
    ct = T // nw
    nch = 4
    cs = ct // nch
    mesh = plsc.VectorSubcoreMesh(core_axis_name="c", subcore_axis_name="s")

    @functools.partial(
        pl.kernel, mesh=mesh,
        out_type=jax.ShapeDtypeStruct((T, D), jnp.float32),
        scratch_types=[pltpu.VMEM((ct,), jnp.int32),
                       pltpu.VMEM((ct,), jnp.int32),
                       pltpu.VMEM((ct, 16), jnp.float32),
                       pltpu.VMEM((ct, 16), jnp.float32),
                       pltpu.VMEM((ct, D), jnp.float32),
                       pltpu.VMEM((ct, D), jnp.float32),
                       pltpu.SemaphoreType.DMA((4,)),
                       pltpu.SemaphoreType.DMA((4,)),
                       pltpu.SemaphoreType.DMA((4,)),
                       pltpu.SemaphoreType.DMA((4,))],
    )
    def k(o_hbm, sl_hbm, w0_hbm, w1_hbm, y_hbm,
          idx0_v, idx1_v, w0_v, w1_v, r0_v, r1_v, sem0, sem1, semy, sems):
        wid = lax.axis_index("s") * nc + lax.axis_index("c")
        tb = wid * ct
        ci0 = pltpu.async_copy(sl_hbm.at[0, pl.ds(tb, ct)], idx0_v, sems.at[0])
        ci1 = pltpu.async_copy(sl_hbm.at[1, pl.ds(tb, ct)], idx1_v, sems.at[1])
        cw0 = pltpu.async_copy(w0_hbm.at[pl.ds(tb, ct)], w0_v, sems.at[2])
        cw1 = pltpu.async_copy(w1_hbm.at[pl.ds(tb, ct)], w1_v, sems.at[3])
        ci0.wait()
        ci1.wait()
        cps = []
        for q in range(nch):
            qs = pl.ds(q * cs, cs)
            cps.append(pltpu.async_copy(o_hbm.at[idx0_v.at[qs]],
                                        r0_v.at[qs], sem0.at[q]))
            cps.append(pltpu.async_copy(o_hbm.at[idx1_v.at[qs]],
                                        r1_v.at[qs], sem1.at[q]))
        cw0.wait()
        cw1.wait()

        def row(r, carry):
            w0 = w0_v[r, :]
            w1 = w1_v[r, :]
            for c in range(D // 16):
                sl = pl.ds(c * 16, 16)
                r0_v[r, sl] = w0 * r0_v[r, sl] + w1 * r1_v[r, sl]
            return carry

        ycps = []
        for q in range(nch):
            cps[2 * q].wait()
            cps[2 * q + 1].wait()
            lax.fori_loop(q * cs, (q + 1) * cs, row, 0)
            qs = pl.ds(q * cs, cs)
            ycps.append(pltpu.async_copy(
                r0_v.at[qs], y_hbm.at[pl.ds(tb + q * cs, cs)], semy.at[q]))
        for cp in ycps:
            cp.wait()

    return k(out_rows, slots, w0x, w1x)


def kernel(hidden_states, Wg, W_in, W_out):
    B, S, Dm = hidden_states.shape
    xf = hidden_states.reshape(T, D)
    sd, sc_, w0x, w1x = _router(xf, Wg)
    buf = _dispatch(xf, sd)
    out = _ffn(buf, W_in, W_out)
    y = _combine(out, sc_, w0x, w1x)
    return y.reshape(B, S, Dm)
